# baseline (speedup 1.0000x reference)
import functools

import numpy as np

import jax
import jax.numpy as jnp
from jax import lax
from jax.experimental import pallas as pl
from jax.experimental.pallas import tpu as pltpu
from jax.experimental.pallas import tpu_sc as plsc

N = 10000
E = 160000
H = 256
HH = 128
G = 128
OUT = 256

NC = 2
NS = 16
EPT = E // NS
KB = 80
NCH = EPT // KB
SUB = 40
NSUB = 2 * NCH
NTC = 62
RCH = 40
NRCH = N // RCH
RK = (NRCH + NS - 1) // NS

_PERM = []
for _c in range(NC):
    for _hi in range(2):
        for _g in range(HH // 32):
            for _l in range(16):
                _PERM.append(_c * HH + 32 * _g + 16 * _hi + _l)
_PERM = tuple(_PERM)


def _make_sc_aggregate(with_gather: bool):
    mesh = plsc.VectorSubcoreMesh(
        core_axis_name="c", subcore_axis_name="s", num_cores=NC, num_subcores=NS)

    scratch = []
    if with_gather:
        scratch += [
            pltpu.VMEM((2, KB), jnp.int32),
            pltpu.VMEM((2, SUB, HH), jnp.float32),
            pltpu.SemaphoreType.DMA((2,)),
            pltpu.SemaphoreType.DMA((2,)),
        ]
    scratch += [
        pltpu.VMEM((4, SUB), jnp.int32),
        pltpu.VMEM((2, KB, HH // 2), jnp.int32),
        pltpu.VMEM((2, SUB, HH), jnp.float32),
        pltpu.VMEM_SHARED((N, HH), jnp.float32),
        pltpu.SemaphoreType.DMA((4,)),
        pltpu.SemaphoreType.DMA((2,)),
        pltpu.SemaphoreType.DMA((2,)),
        pltpu.SemaphoreType.DMA,
    ]

    @functools.partial(
        pl.kernel,
        out_type=jax.ShapeDtypeStruct((NC, N, HH), jnp.float32),
        mesh=mesh,
        scratch_types=scratch,
    )
    def body(*refs):
        if with_gather:
            (msg_hbm, src_hbm, dst_hbm, x_hbm, out_hbm,
             srcb, gb, sidx_sem, g_sem,
             dstb, mb, sb, accum, didx_sem, msg_sem, scat_sem, wsem) = refs
        else:
            (msg_hbm, dst_hbm, out_hbm,
             dstb, mb, sb, accum, didx_sem, msg_sem, scat_sem, wsem) = refs

        cid = lax.axis_index("c")
        sid = lax.axis_index("s")

        def zrow(r, carry):
            for c8 in range(HH // 16):
                sb[0, r, pl.ds(c8 * 16, 16)] = jnp.zeros((16,), jnp.float32)
            return carry
        lax.fori_loop(0, RCH, zrow, 0)
        for k in range(RK):
            ch = sid + k * NS

            @pl.when(ch < NRCH)
            def _z():
                pltpu.async_copy(sb.at[0], accum.at[pl.ds(ch * RCH, RCH)],
                                 wsem)
        for k in range(RK):
            ch = sid + k * NS

            @pl.when(ch < NRCH)
            def _zw():
                pltpu.make_async_copy(
                    sb.at[0], accum.at[pl.ds(ch * RCH, RCH)], wsem).wait()
        plsc.subcore_barrier()

        base0 = sid * EPT

        def a_msg(jc, p):
            base = base0 + jc * KB
            pltpu.async_copy(msg_hbm.at[cid, pl.ds(base, KB)], mb.at[p],
                             msg_sem.at[p])
            if with_gather:
                pltpu.async_copy(src_hbm.at[pl.ds(base, KB)], srcb.at[p],
                                 sidx_sem.at[p])

        def a_didx(s, dslot):
            pltpu.async_copy(dst_hbm.at[pl.ds(base0 + s * SUB, SUB)],
                             dstb.at[dslot], didx_sem.at[dslot])

        def start_gather(jc, p, h):
            if with_gather:
                if h == 0:
                    base = base0 + jc * KB
                    pltpu.make_async_copy(src_hbm.at[pl.ds(base, KB)],
                                          srcb.at[p], sidx_sem.at[p]).wait()
                pltpu.async_copy(
                    x_hbm.at[cid].at[srcb.at[p, pl.ds(h * SUB, SUB)]],
                    gb.at[h], g_sem.at[h])

        def process_sub(jc, p, h, static_jc):
            ss = h
            dslot = 2 * p + h
            sbase = base0 + (2 * jc + h) * SUB

            def _reclaim():
                pltpu.make_async_copy(sb.at[ss],
                                      accum.at[dstb.at[(dslot + 2) % 4]],
                                      scat_sem.at[ss]).wait()
            if static_jc:
                if 2 * jc + h >= 2:
                    _reclaim()
            else:
                pl.when(2 * jc + h >= 2)(_reclaim)
            if static_jc:
                if 2 * jc + h + 2 <= NSUB - 1:
                    a_didx(2 * jc + h + 2, (dslot + 2) % 4)
            else:
                a_didx(2 * jc + h + 2, (dslot + 2) % 4)

            pltpu.make_async_copy(dst_hbm.at[pl.ds(sbase, SUB)],
                                  dstb.at[dslot], didx_sem.at[dslot]).wait()
            if with_gather:
                pltpu.make_async_copy(
                    x_hbm.at[cid].at[srcb.at[p, pl.ds(h * SUB, SUB)]],
                    gb.at[h], g_sem.at[h]).wait()

            def rrow(r, rc):
                for g4 in range(HH // 32):
                    w = mb[p, h * SUB + r, pl.ds(16 * g4, 16)]
                    a = lax.bitcast_convert_type(w << 16, jnp.float32)
                    b = lax.bitcast_convert_type(w & jnp.int32(-65536),
                                                 jnp.float32)
                    if with_gather:
                        a = jnp.maximum(
                            a + gb[h, r, pl.ds(32 * g4, 16)], 0.0)
                        b = jnp.maximum(
                            b + gb[h, r, pl.ds(32 * g4 + 16, 16)], 0.0)
                    sb[ss, r, pl.ds(32 * g4, 16)] = a
                    sb[ss, r, pl.ds(32 * g4 + 16, 16)] = b
                return rc
            lax.fori_loop(0, SUB, rrow, 0)

            pltpu.async_copy(sb.at[ss], accum.at[dstb.at[dslot]],
                             scat_sem.at[ss], add=True)

            if with_gather:
                if static_jc:
                    if 2 * (jc + 1) + h <= NSUB - 1:
                        start_gather(jc + 1, 1 - p, h)
                else:
                    start_gather(jc + 1, 1 - p, h)

        def process_chunk(jc, p, static_jc):
            pltpu.make_async_copy(
                msg_hbm.at[cid, pl.ds(base0 + jc * KB, KB)], mb.at[p],
                msg_sem.at[p]).wait()
            if static_jc:
                if jc + 1 <= NCH - 1:
                    a_msg(jc + 1, 1 - p)
            else:
                a_msg(jc + 1, 1 - p)
            for h in range(2):
                process_sub(jc, p, h, static_jc)

        a_msg(0, 0)
        a_didx(0, 0)
        a_didx(1, 1)
        start_gather(0, 0, 0)
        start_gather(0, 0, 1)

        def outer(t, carry):
            for u in range(2):
                process_chunk(t * 2 + u, u, False)
            return carry
        lax.fori_loop(0, NTC, outer, 0)
        process_chunk(NTC * 2, 0, True)

        for h in range(2):
            pltpu.make_async_copy(sb.at[h], accum.at[dstb.at[h]],
                                  scat_sem.at[h]).wait()

        plsc.subcore_barrier()

        for k in range(RK):
            ch = sid + k * NS

            @pl.when(ch < NRCH)
            def _wb():
                r0 = ch * RCH
                pltpu.async_copy(accum.at[pl.ds(r0, RCH)],
                                 out_hbm.at[cid, pl.ds(r0, RCH)], wsem)
        for k in range(RK):
            ch = sid + k * NS

            @pl.when(ch < NRCH)
            def _wbw():
                r0 = ch * RCH
                pltpu.make_async_copy(accum.at[pl.ds(r0, RCH)],
                                      out_hbm.at[cid, pl.ds(r0, RCH)],
                                      wsem).wait()

    return body


_sc_scatter = _make_sc_aggregate(with_gather=False)
_sc_gather_scatter = _make_sc_aggregate(with_gather=True)


BE = 3200
BN = 1000


def _dotT(a, w):
    return lax.dot_general(a, w, (((1,), (1,)), ((), ())),
                           preferred_element_type=jnp.float32)


def _edge_linear2_body(attr_ref, c_ref, We1_ref, be1_ref, We2_ref, be2_ref,
                       m1_ref, e2_ref):
    at = attr_ref[...]
    dt = lambda w: lax.dot_general(at, w, (((0,), (1,)), ((), ())),
                                   preferred_element_type=jnp.float32)
    def bf16bits(x):
        b = lax.bitcast_convert_type(x, jnp.int32)
        r = b + jnp.int32(0x7FFF) + ((b >> 16) & 1)
        return (r >> 16) & jnp.int32(0xFFFF)

    def pack_half(x, ci):
        lo = x[:, HH * ci:HH * ci + HH // 2]
        hi = x[:, HH * ci + HH // 2:HH * ci + HH]
        return (bf16bits(hi) << 16) | bf16bits(lo)

    e1 = dt(We1_ref[...]) + be1_ref[...] + c_ref[...]
    m1 = jnp.maximum(e1, 0.0)
    e2 = dt(We2_ref[...]) + be2_ref[...]
    m1_ref[0] = pack_half(m1, 0)
    m1_ref[1] = pack_half(m1, 1)
    e2_ref[0] = pack_half(e2, 0)
    e2_ref[1] = pack_half(e2, 1)


def _edge_linear2(edge_attr_t, c, We1, be1, We2, be2):
    grid = (E // BE,)
    full = lambda i: (0, 0)
    return pl.pallas_call(
        _edge_linear2_body,
        grid=grid,
        in_specs=[
            pl.BlockSpec((7, BE), lambda i: (0, i)),
            pl.BlockSpec((1, H), full),
            pl.BlockSpec((H, 7), full),
            pl.BlockSpec((1, H), full),
            pl.BlockSpec((H, 7), full),
            pl.BlockSpec((1, H), full),
        ],
        out_specs=[
            pl.BlockSpec((NC, BE, HH // 2), lambda i: (0, i, 0)),
            pl.BlockSpec((NC, BE, HH // 2), lambda i: (0, i, 0)),
        ],
        out_shape=[
            jax.ShapeDtypeStruct((NC, E, HH // 2), jnp.int32),
            jax.ShapeDtypeStruct((NC, E, HH // 2), jnp.int32),
        ],
    )(edge_attr_t, c, We1, be1, We2, be2)


def _mlp1_body(aggr_ref, c_ref, W11_ref, b11_ref, W12_ref, b12_ref, x1_ref):
    h = jnp.concatenate([aggr_ref[0], aggr_ref[1]], axis=1) + c_ref[...]
    t = jnp.maximum(_dotT(h, W11_ref[...]) + b11_ref[...], 0.0)
    x1 = jnp.maximum(_dotT(t, W12_ref[...]) + b12_ref[...], 0.0)
    x1_ref[0] = x1[:, :HH]
    x1_ref[1] = x1[:, HH:]


def _mlp1(aggr1, c, W11, b11, W12, b12):
    grid = (N // BN,)
    full = lambda i: (0, 0)
    return pl.pallas_call(
        _mlp1_body,
        grid=grid,
        in_specs=[
            pl.BlockSpec((NC, BN, HH), lambda i: (0, i, 0)),
            pl.BlockSpec((1, H), full),
            pl.BlockSpec((H, H), full),
            pl.BlockSpec((1, H), full),
            pl.BlockSpec((H, H), full),
            pl.BlockSpec((1, H), full),
        ],
        out_specs=pl.BlockSpec((NC, BN, HH), lambda i: (0, i, 0)),
        out_shape=jax.ShapeDtypeStruct((NC, N, HH), jnp.float32),
    )(aggr1, c, W11, b11, W12, b12)


def _mlp2_pool_body(x1_ref, aggr_ref, batch_ref,
                    W21_ref, b21_ref, W22_ref, b22_ref, Wl_ref, bl_ref,
                    out_ref, acc, cnt):
    i = pl.program_id(0)

    @pl.when(i == 0)
    def _init():
        acc[...] = jnp.zeros_like(acc)
        cnt[...] = jnp.zeros_like(cnt)

    x1 = jnp.concatenate([x1_ref[0], x1_ref[1]], axis=1)
    h = x1 + jnp.concatenate([aggr_ref[0], aggr_ref[1]], axis=1)
    t = jnp.maximum(_dotT(h, W21_ref[...]) + b21_ref[...], 0.0)
    x2 = _dotT(t, W22_ref[...]) + b22_ref[...]

    b = batch_ref[...]
    gids = lax.broadcasted_iota(jnp.int32, (BN, G), 1)
    onehot = (b == gids).astype(jnp.float32)
    acc[...] += lax.dot_general(onehot, x2, (((0,), (0,)), ((), ())),
                                preferred_element_type=jnp.float32)
    cnt[...] += lax.dot_general(onehot, jnp.ones((BN, H), jnp.float32),
                                (((0,), (0,)), ((), ())),
                                preferred_element_type=jnp.float32)

    @pl.when(i == pl.num_programs(0) - 1)
    def _fin():
        pooled = acc[...] / jnp.maximum(cnt[...], 1.0)
        out_ref[...] = _dotT(pooled, Wl_ref[...]) + bl_ref[...]


def _mlp2_pool(x1, aggr2, batch2d, W21, b21, W22, b22, W_lin, b_lin):
    grid = (N // BN,)
    full = lambda i: (0, 0)
    return pl.pallas_call(
        _mlp2_pool_body,
        grid=grid,
        in_specs=[
            pl.BlockSpec((NC, BN, HH), lambda i: (0, i, 0)),
            pl.BlockSpec((NC, BN, HH), lambda i: (0, i, 0)),
            pl.BlockSpec((BN, 1), lambda i: (i, 0)),
            pl.BlockSpec((H, H), full),
            pl.BlockSpec((1, H), full),
            pl.BlockSpec((H, H), full),
            pl.BlockSpec((1, H), full),
            pl.BlockSpec((OUT, H), full),
            pl.BlockSpec((1, OUT), full),
        ],
        out_specs=pl.BlockSpec((G, OUT), full),
        out_shape=jax.ShapeDtypeStruct((G, OUT), jnp.float32),
        scratch_shapes=[
            pltpu.VMEM((G, H), jnp.float32),
            pltpu.VMEM((G, H), jnp.float32),
        ],
    )(x1, aggr2, batch2d, W21, b21, W22, b22, W_lin, b_lin)


def kernel(x_idx, edge_index, edge_attr, batch, node_emb,
           We1, be1, W11, b11, W12, b12,
           We2, be2, W21, b21, W22, b22,
           W_lin, b_lin):
    c = node_emb.reshape(1, H)
    src = edge_index[0]
    dst = edge_index[1]

    perm = jnp.asarray(_PERM, dtype=jnp.int32)
    We1p = We1[perm]
    be1p = be1[perm].reshape(1, H)
    cp = c[:, perm]
    We2p = We2[perm]
    be2p = be2[perm].reshape(1, H)

    m1, e2 = _edge_linear2(edge_attr.T, cp, We1p, be1p, We2p, be2p)
    aggr1 = _sc_scatter(m1, dst)
    x1 = _mlp1(aggr1, c, W11, b11.reshape(1, H), W12, b12.reshape(1, H))
    aggr2 = _sc_gather_scatter(e2, src, dst, x1)
    out = _mlp2_pool(x1, aggr2, batch.reshape(N, 1),
                     W21, b21.reshape(1, H), W22, b22.reshape(1, H),
                     W_lin, b_lin.reshape(1, OUT))
    return out


# speedup vs baseline: 1.3998x; 1.3998x over previous
import functools

import jax
import jax.numpy as jnp
from jax import lax
from jax.experimental import pallas as pl
from jax.experimental.pallas import tpu as pltpu
from jax.experimental.pallas import tpu_sc as plsc

N = 10000
E = 160000
H = 256
HH = 128
G = 128
OUT = 256

NC = 2
NS = 16
EPT = E // NS
K = 40
NCHUNK = EPT // K
D = 5
NT = NCHUNK // D
RCH = 40
NRCH = N // RCH
RK = (NRCH + NS - 1) // NS


def _make_sc_aggregate(with_gather: bool):
    mesh = plsc.VectorSubcoreMesh(
        core_axis_name="c", subcore_axis_name="s", num_cores=NC, num_subcores=NS)

    scratch = []
    if with_gather:
        scratch += [
            pltpu.VMEM((D, K), jnp.int32),
            pltpu.VMEM((2, K, HH), jnp.float32),
            pltpu.SemaphoreType.DMA((D,)),
            pltpu.SemaphoreType.DMA((2,)),
        ]
    scratch += [
        pltpu.VMEM((D, K), jnp.int32),
        pltpu.VMEM((D, K, HH), jnp.float32),
        pltpu.VMEM_SHARED((N, HH), jnp.float32),
        pltpu.SemaphoreType.DMA((D,)),
        pltpu.SemaphoreType.DMA((D,)),
        pltpu.SemaphoreType.DMA((D,)),
        pltpu.SemaphoreType.DMA,
    ]

    @functools.partial(
        pl.kernel,
        out_type=jax.ShapeDtypeStruct((NC, N, HH), jnp.float32),
        mesh=mesh,
        scratch_types=scratch,
    )
    def body(*refs):
        if with_gather:
            (msg_hbm, src_hbm, dst_hbm, x_hbm, out_hbm,
             srcb, gb, sidx_sem, g_sem,
             dstb, mb, accum, didx_sem, msg_sem, scat_sem, wsem) = refs
        else:
            (msg_hbm, dst_hbm, out_hbm,
             dstb, mb, accum, didx_sem, msg_sem, scat_sem, wsem) = refs

        cid = lax.axis_index("c")
        sid = lax.axis_index("s")

        zbuf = mb.at[0]

        @plsc.parallel_loop(0, RCH, unroll=4)
        def _zrow(r):
            for c8 in range(HH // 16):
                mb[0, r, pl.ds(c8 * 16, 16)] = jnp.zeros((16,), jnp.float32)
        for k in range(RK):
            ch = sid + k * NS

            @pl.when(ch < NRCH)
            def _z():
                pltpu.async_copy(zbuf, accum.at[pl.ds(ch * RCH, RCH)], wsem)
        for k in range(RK):
            ch = sid + k * NS

            @pl.when(ch < NRCH)
            def _zw():
                pltpu.make_async_copy(
                    zbuf, accum.at[pl.ds(ch * RCH, RCH)], wsem).wait()
        plsc.subcore_barrier()

        base0 = sid * EPT

        def start_inputs(jc, p):
            base = base0 + jc * K

            @pl.when(jc >= D)
            def _reclaim():
                pltpu.make_async_copy(
                    mb.at[p], accum.at[dstb.at[p]], scat_sem.at[p]).wait()
            pltpu.async_copy(dst_hbm.at[pl.ds(base, K)], dstb.at[p],
                             didx_sem.at[p])
            pltpu.async_copy(msg_hbm.at[cid, pl.ds(base, K)], mb.at[p],
                             msg_sem.at[p])
            if with_gather:
                pltpu.async_copy(src_hbm.at[pl.ds(base, K)], srcb.at[p],
                                 sidx_sem.at[p])

        def start_gather(jc, p, q):
            base = base0 + jc * K
            pltpu.make_async_copy(src_hbm.at[pl.ds(base, K)], srcb.at[p],
                                  sidx_sem.at[p]).wait()
            pltpu.async_copy(x_hbm.at[cid].at[srcb.at[p]], gb.at[q],
                             g_sem.at[q])

        def process(jc, p, q):
            base = base0 + jc * K
            pltpu.make_async_copy(msg_hbm.at[cid, pl.ds(base, K)], mb.at[p],
                                  msg_sem.at[p]).wait()
            pltpu.make_async_copy(dst_hbm.at[pl.ds(base, K)], dstb.at[p],
                                  didx_sem.at[p]).wait()
            if with_gather:
                pltpu.make_async_copy(x_hbm.at[cid].at[srcb.at[p]], gb.at[q],
                                      g_sem.at[q]).wait()

                def rrow(r, rc):
                    for c8 in range(HH // 16):
                        s = pl.ds(c8 * 16, 16)
                        mb[p, r, s] = jnp.maximum(mb[p, r, s] + gb[q, r, s],
                                                  0.0)
                    return rc
                lax.fori_loop(0, K, rrow, 0)
            pltpu.async_copy(mb.at[p], accum.at[dstb.at[p]], scat_sem.at[p],
                             add=True)

        start_inputs(0, 0)
        start_inputs(1, 1)
        start_inputs(2, 2)
        if with_gather:
            start_gather(0, 0, 0)
            start_gather(1, 1, 1)

        UN = 2 * D

        def outer(t, carry):
            for u in range(UN):
                jc = t * UN + u
                p = u % D
                q = u % 2
                process(jc, p, q)
                if with_gather:
                    @pl.when(jc + 2 < NCHUNK)
                    def _b():
                        start_gather(jc + 2, (u + 2) % D, q)

                @pl.when(jc + 3 < NCHUNK)
                def _a():
                    start_inputs(jc + 3, (u + 3) % D)
            return carry
        lax.fori_loop(0, NCHUNK // UN, outer, 0)

        for p in range(D):
            pltpu.make_async_copy(
                mb.at[p], accum.at[dstb.at[p]], scat_sem.at[p]).wait()

        plsc.subcore_barrier()

        for k in range(RK):
            ch = sid + k * NS

            @pl.when(ch < NRCH)
            def _wb():
                r0 = ch * RCH
                pltpu.async_copy(accum.at[pl.ds(r0, RCH)],
                                 out_hbm.at[cid, pl.ds(r0, RCH)], wsem)
        for k in range(RK):
            ch = sid + k * NS

            @pl.when(ch < NRCH)
            def _wbw():
                r0 = ch * RCH
                pltpu.make_async_copy(accum.at[pl.ds(r0, RCH)],
                                      out_hbm.at[cid, pl.ds(r0, RCH)],
                                      wsem).wait()

    return body


_sc_scatter = _make_sc_aggregate(with_gather=False)
_sc_gather_scatter = _make_sc_aggregate(with_gather=True)


BE = 3200
BN = 1000


def _dotT(a, w):
    return lax.dot_general(a, w, (((1,), (1,)), ((), ())),
                           preferred_element_type=jnp.float32)


def _edge_linear2_body(attr_ref, c_ref, We1_ref, be1_ref, We2_ref, be2_ref,
                       m1_ref, e2_ref):
    at = attr_ref[...]
    dt = lambda w: lax.dot_general(at, w, (((0,), (1,)), ((), ())),
                                   preferred_element_type=jnp.float32)
    e1 = dt(We1_ref[...]) + be1_ref[...] + c_ref[...]
    m1 = jnp.maximum(e1, 0.0)
    e2 = dt(We2_ref[...]) + be2_ref[...]
    m1_ref[0] = m1[:, :HH]
    m1_ref[1] = m1[:, HH:]
    e2_ref[0] = e2[:, :HH]
    e2_ref[1] = e2[:, HH:]


def _edge_linear2(edge_attr_t, c, We1, be1, We2, be2):
    grid = (E // BE,)
    full = lambda i: (0, 0)
    return pl.pallas_call(
        _edge_linear2_body,
        grid=grid,
        in_specs=[
            pl.BlockSpec((7, BE), lambda i: (0, i)),
            pl.BlockSpec((1, H), full),
            pl.BlockSpec((H, 7), full),
            pl.BlockSpec((1, H), full),
            pl.BlockSpec((H, 7), full),
            pl.BlockSpec((1, H), full),
        ],
        out_specs=[
            pl.BlockSpec((NC, BE, HH), lambda i: (0, i, 0)),
            pl.BlockSpec((NC, BE, HH), lambda i: (0, i, 0)),
        ],
        out_shape=[
            jax.ShapeDtypeStruct((NC, E, HH), jnp.float32),
            jax.ShapeDtypeStruct((NC, E, HH), jnp.float32),
        ],
    )(edge_attr_t, c, We1, be1, We2, be2)


def _mlp1_body(aggr_ref, c_ref, W11_ref, b11_ref, W12_ref, b12_ref, x1_ref):
    h = jnp.concatenate([aggr_ref[0], aggr_ref[1]], axis=1) + c_ref[...]
    t = jnp.maximum(_dotT(h, W11_ref[...]) + b11_ref[...], 0.0)
    x1 = jnp.maximum(_dotT(t, W12_ref[...]) + b12_ref[...], 0.0)
    x1_ref[0] = x1[:, :HH]
    x1_ref[1] = x1[:, HH:]


def _mlp1(aggr1, c, W11, b11, W12, b12):
    grid = (N // BN,)
    full = lambda i: (0, 0)
    return pl.pallas_call(
        _mlp1_body,
        grid=grid,
        in_specs=[
            pl.BlockSpec((NC, BN, HH), lambda i: (0, i, 0)),
            pl.BlockSpec((1, H), full),
            pl.BlockSpec((H, H), full),
            pl.BlockSpec((1, H), full),
            pl.BlockSpec((H, H), full),
            pl.BlockSpec((1, H), full),
        ],
        out_specs=pl.BlockSpec((NC, BN, HH), lambda i: (0, i, 0)),
        out_shape=jax.ShapeDtypeStruct((NC, N, HH), jnp.float32),
    )(aggr1, c, W11, b11, W12, b12)


def _mlp2_pool_body(x1_ref, aggr_ref, batch_ref,
                    W21_ref, b21_ref, W22_ref, b22_ref, Wl_ref, bl_ref,
                    out_ref, acc, cnt):
    i = pl.program_id(0)

    @pl.when(i == 0)
    def _init():
        acc[...] = jnp.zeros_like(acc)
        cnt[...] = jnp.zeros_like(cnt)

    x1 = jnp.concatenate([x1_ref[0], x1_ref[1]], axis=1)
    h = x1 + jnp.concatenate([aggr_ref[0], aggr_ref[1]], axis=1)
    t = jnp.maximum(_dotT(h, W21_ref[...]) + b21_ref[...], 0.0)
    x2 = _dotT(t, W22_ref[...]) + b22_ref[...]

    b = batch_ref[...]
    gids = lax.broadcasted_iota(jnp.int32, (BN, G), 1)
    onehot = (b == gids).astype(jnp.float32)
    acc[...] += lax.dot_general(onehot, x2, (((0,), (0,)), ((), ())),
                                preferred_element_type=jnp.float32)
    cnt[...] += lax.dot_general(onehot, jnp.ones((BN, H), jnp.float32),
                                (((0,), (0,)), ((), ())),
                                preferred_element_type=jnp.float32)

    @pl.when(i == pl.num_programs(0) - 1)
    def _fin():
        pooled = acc[...] / jnp.maximum(cnt[...], 1.0)
        out_ref[...] = _dotT(pooled, Wl_ref[...]) + bl_ref[...]


def _mlp2_pool(x1, aggr2, batch2d, W21, b21, W22, b22, W_lin, b_lin):
    grid = (N // BN,)
    full = lambda i: (0, 0)
    return pl.pallas_call(
        _mlp2_pool_body,
        grid=grid,
        in_specs=[
            pl.BlockSpec((NC, BN, HH), lambda i: (0, i, 0)),
            pl.BlockSpec((NC, BN, HH), lambda i: (0, i, 0)),
            pl.BlockSpec((BN, 1), lambda i: (i, 0)),
            pl.BlockSpec((H, H), full),
            pl.BlockSpec((1, H), full),
            pl.BlockSpec((H, H), full),
            pl.BlockSpec((1, H), full),
            pl.BlockSpec((OUT, H), full),
            pl.BlockSpec((1, OUT), full),
        ],
        out_specs=pl.BlockSpec((G, OUT), full),
        out_shape=jax.ShapeDtypeStruct((G, OUT), jnp.float32),
        scratch_shapes=[
            pltpu.VMEM((G, H), jnp.float32),
            pltpu.VMEM((G, H), jnp.float32),
        ],
    )(x1, aggr2, batch2d, W21, b21, W22, b22, W_lin, b_lin)


def kernel(x_idx, edge_index, edge_attr, batch, node_emb,
           We1, be1, W11, b11, W12, b12,
           We2, be2, W21, b21, W22, b22,
           W_lin, b_lin):
    c = node_emb.reshape(1, H)
    src = edge_index[0]
    dst = edge_index[1]

    m1, e2 = _edge_linear2(edge_attr.T, c, We1, be1.reshape(1, H),
                           We2, be2.reshape(1, H))
    aggr1 = _sc_scatter(m1, dst)
    x1 = _mlp1(aggr1, c, W11, b11.reshape(1, H), W12, b12.reshape(1, H))
    aggr2 = _sc_gather_scatter(e2, src, dst, x1)
    out = _mlp2_pool(x1, aggr2, batch.reshape(N, 1),
                     W21, b21.reshape(1, H), W22, b22.reshape(1, H),
                     W_lin, b_lin.reshape(1, OUT))
    return out


# speedup vs baseline: 1.4280x; 1.0202x over previous
import functools

import jax
import jax.numpy as jnp
from jax import lax
from jax.experimental import pallas as pl
from jax.experimental.pallas import tpu as pltpu
from jax.experimental.pallas import tpu_sc as plsc

N = 10000
E = 160000
H = 256
HH = 128
G = 128
OUT = 256

NC = 2
NS = 16
EPT = E // NS
K = 40
NCHUNK = EPT // K
D = 5
NT = NCHUNK // D
RCH = 40
NRCH = N // RCH
RK = (NRCH + NS - 1) // NS


def _make_sc_aggregate(with_gather: bool):
    mesh = plsc.VectorSubcoreMesh(
        core_axis_name="c", subcore_axis_name="s", num_cores=NC, num_subcores=NS)

    scratch = []
    if with_gather:
        scratch += [
            pltpu.VMEM((D, K), jnp.int32),
            pltpu.VMEM((2, K, HH), jnp.float32),
            pltpu.SemaphoreType.DMA((D,)),
            pltpu.SemaphoreType.DMA((2,)),
        ]
    scratch += [
        pltpu.VMEM((D, K), jnp.int32),
        pltpu.VMEM((D, K, HH), jnp.float32),
        pltpu.VMEM_SHARED((N, HH), jnp.float32),
        pltpu.SemaphoreType.DMA((D,)),
        pltpu.SemaphoreType.DMA((D,)),
        pltpu.SemaphoreType.DMA((D,)),
        pltpu.SemaphoreType.DMA,
    ]

    @functools.partial(
        pl.kernel,
        out_type=jax.ShapeDtypeStruct((NC, N, HH), jnp.float32),
        mesh=mesh,
        scratch_types=scratch,
    )
    def body(*refs):
        if with_gather:
            (msg_hbm, src_hbm, dst_hbm, x_hbm, out_hbm,
             srcb, gb, sidx_sem, g_sem,
             dstb, mb, accum, didx_sem, msg_sem, scat_sem, wsem) = refs
        else:
            (msg_hbm, dst_hbm, out_hbm,
             dstb, mb, accum, didx_sem, msg_sem, scat_sem, wsem) = refs

        cid = lax.axis_index("c")
        sid = lax.axis_index("s")

        zbuf = mb.at[0]

        @plsc.parallel_loop(0, RCH, unroll=4)
        def _zrow(r):
            for c8 in range(HH // 16):
                mb[0, r, pl.ds(c8 * 16, 16)] = jnp.zeros((16,), jnp.float32)
        for k in range(RK):
            ch = sid + k * NS

            @pl.when(ch < NRCH)
            def _z():
                pltpu.async_copy(zbuf, accum.at[pl.ds(ch * RCH, RCH)], wsem)
        for k in range(RK):
            ch = sid + k * NS

            @pl.when(ch < NRCH)
            def _zw():
                pltpu.make_async_copy(
                    zbuf, accum.at[pl.ds(ch * RCH, RCH)], wsem).wait()
        plsc.subcore_barrier()

        base0 = sid * EPT

        def start_inputs(jc, p):
            base = base0 + jc * K

            @pl.when(jc >= D)
            def _reclaim():
                pltpu.make_async_copy(
                    mb.at[p], accum.at[dstb.at[p]], scat_sem.at[p]).wait()
            pltpu.async_copy(dst_hbm.at[pl.ds(base, K)], dstb.at[p],
                             didx_sem.at[p])
            pltpu.async_copy(msg_hbm.at[cid, pl.ds(base, K)], mb.at[p],
                             msg_sem.at[p])
            if with_gather:
                pltpu.async_copy(src_hbm.at[pl.ds(base, K)], srcb.at[p],
                                 sidx_sem.at[p])

        def start_gather(jc, p, q):
            base = base0 + jc * K
            pltpu.make_async_copy(src_hbm.at[pl.ds(base, K)], srcb.at[p],
                                  sidx_sem.at[p]).wait()
            pltpu.async_copy(x_hbm.at[cid].at[srcb.at[p]], gb.at[q],
                             g_sem.at[q])

        def process(jc, p, q):
            base = base0 + jc * K
            pltpu.make_async_copy(msg_hbm.at[cid, pl.ds(base, K)], mb.at[p],
                                  msg_sem.at[p]).wait()
            pltpu.make_async_copy(dst_hbm.at[pl.ds(base, K)], dstb.at[p],
                                  didx_sem.at[p]).wait()
            if with_gather:
                pltpu.make_async_copy(x_hbm.at[cid].at[srcb.at[p]], gb.at[q],
                                      g_sem.at[q]).wait()

                def rrow(r, rc):
                    for c8 in range(HH // 16):
                        s = pl.ds(c8 * 16, 16)
                        mb[p, r, s] = jnp.maximum(mb[p, r, s] + gb[q, r, s],
                                                  0.0)
                    return rc
                lax.fori_loop(0, K, rrow, 0)
            pltpu.async_copy(mb.at[p], accum.at[dstb.at[p]], scat_sem.at[p],
                             add=True)

        start_inputs(0, 0)
        start_inputs(1, 1)
        start_inputs(2, 2)
        start_inputs(3, 3)
        if with_gather:
            start_gather(0, 0, 0)
            start_gather(1, 1, 1)

        UN = 2 * D

        def outer(t, carry):
            for u in range(UN):
                jc = t * UN + u
                p = u % D
                q = u % 2
                process(jc, p, q)
                if with_gather:
                    @pl.when(jc + 2 < NCHUNK)
                    def _b():
                        start_gather(jc + 2, (u + 2) % D, q)

                @pl.when(jc + 4 < NCHUNK)
                def _a():
                    start_inputs(jc + 4, (u + 4) % D)
            return carry
        lax.fori_loop(0, NCHUNK // UN, outer, 0)

        for p in range(D):
            pltpu.make_async_copy(
                mb.at[p], accum.at[dstb.at[p]], scat_sem.at[p]).wait()

        plsc.subcore_barrier()

        for k in range(RK):
            ch = sid + k * NS

            @pl.when(ch < NRCH)
            def _wb():
                r0 = ch * RCH
                pltpu.async_copy(accum.at[pl.ds(r0, RCH)],
                                 out_hbm.at[cid, pl.ds(r0, RCH)], wsem)
        for k in range(RK):
            ch = sid + k * NS

            @pl.when(ch < NRCH)
            def _wbw():
                r0 = ch * RCH
                pltpu.make_async_copy(accum.at[pl.ds(r0, RCH)],
                                      out_hbm.at[cid, pl.ds(r0, RCH)],
                                      wsem).wait()

    return body


_sc_scatter = _make_sc_aggregate(with_gather=False)
_sc_gather_scatter = _make_sc_aggregate(with_gather=True)


BE = 3200
BN = 1000


def _dotT(a, w):
    return lax.dot_general(a, w, (((1,), (1,)), ((), ())),
                           preferred_element_type=jnp.float32)


def _edge_linear2_body(attr_ref, c_ref, We1_ref, be1_ref, We2_ref, be2_ref,
                       m1_ref, e2_ref):
    at = attr_ref[...]
    dt = lambda w: lax.dot_general(at, w, (((0,), (1,)), ((), ())),
                                   preferred_element_type=jnp.float32)
    e1 = dt(We1_ref[...]) + be1_ref[...] + c_ref[...]
    m1 = jnp.maximum(e1, 0.0)
    e2 = dt(We2_ref[...]) + be2_ref[...]
    m1_ref[0] = m1[:, :HH]
    m1_ref[1] = m1[:, HH:]
    e2_ref[0] = e2[:, :HH]
    e2_ref[1] = e2[:, HH:]


def _edge_linear2(edge_attr_t, c, We1, be1, We2, be2):
    grid = (E // BE,)
    full = lambda i: (0, 0)
    return pl.pallas_call(
        _edge_linear2_body,
        grid=grid,
        in_specs=[
            pl.BlockSpec((7, BE), lambda i: (0, i)),
            pl.BlockSpec((1, H), full),
            pl.BlockSpec((H, 7), full),
            pl.BlockSpec((1, H), full),
            pl.BlockSpec((H, 7), full),
            pl.BlockSpec((1, H), full),
        ],
        out_specs=[
            pl.BlockSpec((NC, BE, HH), lambda i: (0, i, 0)),
            pl.BlockSpec((NC, BE, HH), lambda i: (0, i, 0)),
        ],
        out_shape=[
            jax.ShapeDtypeStruct((NC, E, HH), jnp.float32),
            jax.ShapeDtypeStruct((NC, E, HH), jnp.float32),
        ],
    )(edge_attr_t, c, We1, be1, We2, be2)


def _mlp1_body(aggr_ref, c_ref, W11_ref, b11_ref, W12_ref, b12_ref, x1_ref):
    h = jnp.concatenate([aggr_ref[0], aggr_ref[1]], axis=1) + c_ref[...]
    t = jnp.maximum(_dotT(h, W11_ref[...]) + b11_ref[...], 0.0)
    x1 = jnp.maximum(_dotT(t, W12_ref[...]) + b12_ref[...], 0.0)
    x1_ref[0] = x1[:, :HH]
    x1_ref[1] = x1[:, HH:]


def _mlp1(aggr1, c, W11, b11, W12, b12):
    grid = (N // BN,)
    full = lambda i: (0, 0)
    return pl.pallas_call(
        _mlp1_body,
        grid=grid,
        in_specs=[
            pl.BlockSpec((NC, BN, HH), lambda i: (0, i, 0)),
            pl.BlockSpec((1, H), full),
            pl.BlockSpec((H, H), full),
            pl.BlockSpec((1, H), full),
            pl.BlockSpec((H, H), full),
            pl.BlockSpec((1, H), full),
        ],
        out_specs=pl.BlockSpec((NC, BN, HH), lambda i: (0, i, 0)),
        out_shape=jax.ShapeDtypeStruct((NC, N, HH), jnp.float32),
    )(aggr1, c, W11, b11, W12, b12)


def _mlp2_pool_body(x1_ref, aggr_ref, batch_ref,
                    W21_ref, b21_ref, W22_ref, b22_ref, Wl_ref, bl_ref,
                    out_ref, acc, cnt):
    i = pl.program_id(0)

    @pl.when(i == 0)
    def _init():
        acc[...] = jnp.zeros_like(acc)
        cnt[...] = jnp.zeros_like(cnt)

    x1 = jnp.concatenate([x1_ref[0], x1_ref[1]], axis=1)
    h = x1 + jnp.concatenate([aggr_ref[0], aggr_ref[1]], axis=1)
    t = jnp.maximum(_dotT(h, W21_ref[...]) + b21_ref[...], 0.0)
    x2 = _dotT(t, W22_ref[...]) + b22_ref[...]

    b = batch_ref[...]
    gids = lax.broadcasted_iota(jnp.int32, (BN, G), 1)
    onehot = (b == gids).astype(jnp.float32)
    acc[...] += lax.dot_general(onehot, x2, (((0,), (0,)), ((), ())),
                                preferred_element_type=jnp.float32)
    cnt[...] += lax.dot_general(onehot, jnp.ones((BN, H), jnp.float32),
                                (((0,), (0,)), ((), ())),
                                preferred_element_type=jnp.float32)

    @pl.when(i == pl.num_programs(0) - 1)
    def _fin():
        pooled = acc[...] / jnp.maximum(cnt[...], 1.0)
        out_ref[...] = _dotT(pooled, Wl_ref[...]) + bl_ref[...]


def _mlp2_pool(x1, aggr2, batch2d, W21, b21, W22, b22, W_lin, b_lin):
    grid = (N // BN,)
    full = lambda i: (0, 0)
    return pl.pallas_call(
        _mlp2_pool_body,
        grid=grid,
        in_specs=[
            pl.BlockSpec((NC, BN, HH), lambda i: (0, i, 0)),
            pl.BlockSpec((NC, BN, HH), lambda i: (0, i, 0)),
            pl.BlockSpec((BN, 1), lambda i: (i, 0)),
            pl.BlockSpec((H, H), full),
            pl.BlockSpec((1, H), full),
            pl.BlockSpec((H, H), full),
            pl.BlockSpec((1, H), full),
            pl.BlockSpec((OUT, H), full),
            pl.BlockSpec((1, OUT), full),
        ],
        out_specs=pl.BlockSpec((G, OUT), full),
        out_shape=jax.ShapeDtypeStruct((G, OUT), jnp.float32),
        scratch_shapes=[
            pltpu.VMEM((G, H), jnp.float32),
            pltpu.VMEM((G, H), jnp.float32),
        ],
    )(x1, aggr2, batch2d, W21, b21, W22, b22, W_lin, b_lin)


def kernel(x_idx, edge_index, edge_attr, batch, node_emb,
           We1, be1, W11, b11, W12, b12,
           We2, be2, W21, b21, W22, b22,
           W_lin, b_lin):
    c = node_emb.reshape(1, H)
    src = edge_index[0]
    dst = edge_index[1]

    m1, e2 = _edge_linear2(edge_attr.T, c, We1, be1.reshape(1, H),
                           We2, be2.reshape(1, H))
    aggr1 = _sc_scatter(m1, dst)
    x1 = _mlp1(aggr1, c, W11, b11.reshape(1, H), W12, b12.reshape(1, H))
    aggr2 = _sc_gather_scatter(e2, src, dst, x1)
    out = _mlp2_pool(x1, aggr2, batch.reshape(N, 1),
                     W21, b21.reshape(1, H), W22, b22.reshape(1, H),
                     W_lin, b_lin.reshape(1, OUT))
    return out


# speedup vs baseline: 1.4338x; 1.0040x over previous
import functools

import jax
import jax.numpy as jnp
from jax import lax
from jax.experimental import pallas as pl
from jax.experimental.pallas import tpu as pltpu
from jax.experimental.pallas import tpu_sc as plsc

N = 10000
E = 160000
H = 256
HH = 128
G = 128
OUT = 256

NC = 2
NS = 16
EPT = E // NS
K = 40
NCHUNK = EPT // K
D = 5
NT = NCHUNK // D
RCH = 40
NRCH = N // RCH
RK = (NRCH + NS - 1) // NS


def _make_sc_aggregate(with_gather: bool):
    mesh = plsc.VectorSubcoreMesh(
        core_axis_name="c", subcore_axis_name="s", num_cores=NC, num_subcores=NS)

    scratch = []
    if with_gather:
        scratch += [
            pltpu.VMEM((D, K), jnp.int32),
            pltpu.VMEM((2, K, HH), jnp.float32),
            pltpu.SemaphoreType.DMA((D,)),
            pltpu.SemaphoreType.DMA((2,)),
        ]
    scratch += [
        pltpu.VMEM((D, K), jnp.int32),
        pltpu.VMEM((D, K, HH), jnp.float32),
        pltpu.VMEM_SHARED((N, HH), jnp.float32),
        pltpu.SemaphoreType.DMA((D,)),
        pltpu.SemaphoreType.DMA((D,)),
        pltpu.SemaphoreType.DMA((D,)),
        pltpu.SemaphoreType.DMA,
    ]

    @functools.partial(
        pl.kernel,
        out_type=jax.ShapeDtypeStruct((NC, N, HH), jnp.float32),
        mesh=mesh,
        scratch_types=scratch,
    )
    def body(*refs):
        if with_gather:
            (msg_hbm, src_hbm, dst_hbm, x_hbm, out_hbm,
             srcb, gb, sidx_sem, g_sem,
             dstb, mb, accum, didx_sem, msg_sem, scat_sem, wsem) = refs
        else:
            (msg_hbm, dst_hbm, out_hbm,
             dstb, mb, accum, didx_sem, msg_sem, scat_sem, wsem) = refs

        cid = lax.axis_index("c")
        sid = lax.axis_index("s")

        zbuf = mb.at[0]

        @plsc.parallel_loop(0, RCH, unroll=4)
        def _zrow(r):
            for c8 in range(HH // 16):
                mb[0, r, pl.ds(c8 * 16, 16)] = jnp.zeros((16,), jnp.float32)
        for k in range(RK):
            ch = sid + k * NS

            @pl.when(ch < NRCH)
            def _z():
                pltpu.async_copy(zbuf, accum.at[pl.ds(ch * RCH, RCH)], wsem)
        for k in range(RK):
            ch = sid + k * NS

            @pl.when(ch < NRCH)
            def _zw():
                pltpu.make_async_copy(
                    zbuf, accum.at[pl.ds(ch * RCH, RCH)], wsem).wait()
        plsc.subcore_barrier()

        base0 = sid * EPT

        def start_inputs(jc, p):
            base = base0 + jc * K

            @pl.when(jc >= D)
            def _reclaim():
                pltpu.make_async_copy(
                    mb.at[p], accum.at[dstb.at[p]], scat_sem.at[p]).wait()
            pltpu.async_copy(dst_hbm.at[pl.ds(base, K)], dstb.at[p],
                             didx_sem.at[p])
            pltpu.async_copy(msg_hbm.at[cid, pl.ds(base, K)], mb.at[p],
                             msg_sem.at[p])
            if with_gather:
                pltpu.async_copy(src_hbm.at[pl.ds(base, K)], srcb.at[p],
                                 sidx_sem.at[p])

        def start_gather(jc, p, q):
            base = base0 + jc * K
            pltpu.make_async_copy(src_hbm.at[pl.ds(base, K)], srcb.at[p],
                                  sidx_sem.at[p]).wait()
            pltpu.async_copy(x_hbm.at[cid].at[srcb.at[p]], gb.at[q],
                             g_sem.at[q])

        def process(jc, p, q):
            base = base0 + jc * K
            pltpu.make_async_copy(msg_hbm.at[cid, pl.ds(base, K)], mb.at[p],
                                  msg_sem.at[p]).wait()
            pltpu.make_async_copy(dst_hbm.at[pl.ds(base, K)], dstb.at[p],
                                  didx_sem.at[p]).wait()
            if with_gather:
                pltpu.make_async_copy(x_hbm.at[cid].at[srcb.at[p]], gb.at[q],
                                      g_sem.at[q]).wait()

                def rrow(r, rc):
                    for c8 in range(HH // 16):
                        s = pl.ds(c8 * 16, 16)
                        mb[p, r, s] = jnp.maximum(mb[p, r, s] + gb[q, r, s],
                                                  0.0)
                    return rc
                lax.fori_loop(0, K, rrow, 0)
            pltpu.async_copy(mb.at[p], accum.at[dstb.at[p]], scat_sem.at[p],
                             add=True)

        start_inputs(0, 0)
        start_inputs(1, 1)
        start_inputs(2, 2)
        start_inputs(3, 3)
        if with_gather:
            start_gather(0, 0, 0)
            start_gather(1, 1, 1)

        UN = 2 * D

        def outer(t, carry):
            for u in range(UN):
                jc = t * UN + u
                p = u % D
                q = u % 2
                process(jc, p, q)
                if with_gather:
                    @pl.when(jc + 2 < NCHUNK)
                    def _b():
                        start_gather(jc + 2, (u + 2) % D, q)

                @pl.when(jc + 4 < NCHUNK)
                def _a():
                    start_inputs(jc + 4, (u + 4) % D)
            return carry
        lax.fori_loop(0, NCHUNK // UN, outer, 0)

        for p in range(D):
            pltpu.make_async_copy(
                mb.at[p], accum.at[dstb.at[p]], scat_sem.at[p]).wait()

        plsc.subcore_barrier()

        for k in range(RK):
            ch = sid + k * NS

            @pl.when(ch < NRCH)
            def _wb():
                r0 = ch * RCH
                pltpu.async_copy(accum.at[pl.ds(r0, RCH)],
                                 out_hbm.at[cid, pl.ds(r0, RCH)], wsem)
        for k in range(RK):
            ch = sid + k * NS

            @pl.when(ch < NRCH)
            def _wbw():
                r0 = ch * RCH
                pltpu.make_async_copy(accum.at[pl.ds(r0, RCH)],
                                      out_hbm.at[cid, pl.ds(r0, RCH)],
                                      wsem).wait()

    return body


_sc_scatter = _make_sc_aggregate(with_gather=False)
_sc_gather_scatter = _make_sc_aggregate(with_gather=True)


BE = 6400
BN = 1000


def _dotT(a, w):
    return lax.dot_general(a, w, (((1,), (1,)), ((), ())),
                           preferred_element_type=jnp.float32)


def _edge_linear2_body(attr_ref, c_ref, We1_ref, be1_ref, We2_ref, be2_ref,
                       m1_ref, e2_ref):
    at = attr_ref[...]
    dt = lambda w: lax.dot_general(at, w, (((0,), (1,)), ((), ())),
                                   preferred_element_type=jnp.float32)
    e1 = dt(We1_ref[...]) + be1_ref[...] + c_ref[...]
    m1 = jnp.maximum(e1, 0.0)
    e2 = dt(We2_ref[...]) + be2_ref[...]
    m1_ref[0] = m1[:, :HH]
    m1_ref[1] = m1[:, HH:]
    e2_ref[0] = e2[:, :HH]
    e2_ref[1] = e2[:, HH:]


def _edge_linear2(edge_attr_t, c, We1, be1, We2, be2):
    grid = (E // BE,)
    full = lambda i: (0, 0)
    return pl.pallas_call(
        _edge_linear2_body,
        grid=grid,
        in_specs=[
            pl.BlockSpec((7, BE), lambda i: (0, i)),
            pl.BlockSpec((1, H), full),
            pl.BlockSpec((H, 7), full),
            pl.BlockSpec((1, H), full),
            pl.BlockSpec((H, 7), full),
            pl.BlockSpec((1, H), full),
        ],
        out_specs=[
            pl.BlockSpec((NC, BE, HH), lambda i: (0, i, 0)),
            pl.BlockSpec((NC, BE, HH), lambda i: (0, i, 0)),
        ],
        out_shape=[
            jax.ShapeDtypeStruct((NC, E, HH), jnp.float32),
            jax.ShapeDtypeStruct((NC, E, HH), jnp.float32),
        ],
    )(edge_attr_t, c, We1, be1, We2, be2)


def _mlp1_body(aggr_ref, c_ref, W11_ref, b11_ref, W12_ref, b12_ref, x1_ref):
    h = jnp.concatenate([aggr_ref[0], aggr_ref[1]], axis=1) + c_ref[...]
    t = jnp.maximum(_dotT(h, W11_ref[...]) + b11_ref[...], 0.0)
    x1 = jnp.maximum(_dotT(t, W12_ref[...]) + b12_ref[...], 0.0)
    x1_ref[0] = x1[:, :HH]
    x1_ref[1] = x1[:, HH:]


def _mlp1(aggr1, c, W11, b11, W12, b12):
    grid = (N // BN,)
    full = lambda i: (0, 0)
    return pl.pallas_call(
        _mlp1_body,
        grid=grid,
        in_specs=[
            pl.BlockSpec((NC, BN, HH), lambda i: (0, i, 0)),
            pl.BlockSpec((1, H), full),
            pl.BlockSpec((H, H), full),
            pl.BlockSpec((1, H), full),
            pl.BlockSpec((H, H), full),
            pl.BlockSpec((1, H), full),
        ],
        out_specs=pl.BlockSpec((NC, BN, HH), lambda i: (0, i, 0)),
        out_shape=jax.ShapeDtypeStruct((NC, N, HH), jnp.float32),
    )(aggr1, c, W11, b11, W12, b12)


def _mlp2_pool_body(x1_ref, aggr_ref, batch_ref,
                    W21_ref, b21_ref, W22_ref, b22_ref, Wl_ref, bl_ref,
                    out_ref, acc, cnt):
    i = pl.program_id(0)

    @pl.when(i == 0)
    def _init():
        acc[...] = jnp.zeros_like(acc)
        cnt[...] = jnp.zeros_like(cnt)

    x1 = jnp.concatenate([x1_ref[0], x1_ref[1]], axis=1)
    h = x1 + jnp.concatenate([aggr_ref[0], aggr_ref[1]], axis=1)
    t = jnp.maximum(_dotT(h, W21_ref[...]) + b21_ref[...], 0.0)
    x2 = _dotT(t, W22_ref[...]) + b22_ref[...]

    b = batch_ref[...]
    gids = lax.broadcasted_iota(jnp.int32, (BN, G), 1)
    onehot = (b == gids).astype(jnp.float32)
    acc[...] += lax.dot_general(onehot, x2, (((0,), (0,)), ((), ())),
                                preferred_element_type=jnp.float32)
    cnt[...] += lax.dot_general(onehot, jnp.ones((BN, H), jnp.float32),
                                (((0,), (0,)), ((), ())),
                                preferred_element_type=jnp.float32)

    @pl.when(i == pl.num_programs(0) - 1)
    def _fin():
        pooled = acc[...] / jnp.maximum(cnt[...], 1.0)
        out_ref[...] = _dotT(pooled, Wl_ref[...]) + bl_ref[...]


def _mlp2_pool(x1, aggr2, batch2d, W21, b21, W22, b22, W_lin, b_lin):
    grid = (N // BN,)
    full = lambda i: (0, 0)
    return pl.pallas_call(
        _mlp2_pool_body,
        grid=grid,
        in_specs=[
            pl.BlockSpec((NC, BN, HH), lambda i: (0, i, 0)),
            pl.BlockSpec((NC, BN, HH), lambda i: (0, i, 0)),
            pl.BlockSpec((BN, 1), lambda i: (i, 0)),
            pl.BlockSpec((H, H), full),
            pl.BlockSpec((1, H), full),
            pl.BlockSpec((H, H), full),
            pl.BlockSpec((1, H), full),
            pl.BlockSpec((OUT, H), full),
            pl.BlockSpec((1, OUT), full),
        ],
        out_specs=pl.BlockSpec((G, OUT), full),
        out_shape=jax.ShapeDtypeStruct((G, OUT), jnp.float32),
        scratch_shapes=[
            pltpu.VMEM((G, H), jnp.float32),
            pltpu.VMEM((G, H), jnp.float32),
        ],
    )(x1, aggr2, batch2d, W21, b21, W22, b22, W_lin, b_lin)


def kernel(x_idx, edge_index, edge_attr, batch, node_emb,
           We1, be1, W11, b11, W12, b12,
           We2, be2, W21, b21, W22, b22,
           W_lin, b_lin):
    c = node_emb.reshape(1, H)
    src = edge_index[0]
    dst = edge_index[1]

    m1, e2 = _edge_linear2(edge_attr.T, c, We1, be1.reshape(1, H),
                           We2, be2.reshape(1, H))
    aggr1 = _sc_scatter(m1, dst)
    x1 = _mlp1(aggr1, c, W11, b11.reshape(1, H), W12, b12.reshape(1, H))
    aggr2 = _sc_gather_scatter(e2, src, dst, x1)
    out = _mlp2_pool(x1, aggr2, batch.reshape(N, 1),
                     W21, b21.reshape(1, H), W22, b22.reshape(1, H),
                     W_lin, b_lin.reshape(1, OUT))
    return out
